# baseline (speedup 1.0000x reference)
import functools

import jax
import jax.numpy as jnp
from jax import lax
from jax.experimental import pallas as pl
from jax.experimental.pallas import tpu as pltpu
from jax.experimental.pallas import tpu_sc as plsc

NC = 2
NS = 16
NW = NC * NS
L = 16

B = 16384
D = 64
CHUNK = 128
BPW = B // NW
NCHUNK = BPW // CHUNK

HID1 = 256
HID2 = 128
BLK = 2048


def _gather_body(ut_hbm, mt_hbm, uid_hbm, mid_hbm, u_out, m_out,
                 uidx_v, midx_v, urows_v, mrows_v, sem):
    wid = lax.axis_index("s") * NC + lax.axis_index("c")
    base = wid * BPW
    pltpu.sync_copy(uid_hbm.at[pl.ds(wid * NCHUNK, NCHUNK)], uidx_v)
    pltpu.sync_copy(mid_hbm.at[pl.ds(wid * NCHUNK, NCHUNK)], midx_v)
    copies = []
    for j in range(NCHUNK):
        copies.append(pltpu.async_copy(
            ut_hbm.at[uidx_v.at[j]], urows_v.at[pl.ds(j * CHUNK, CHUNK)], sem))
        copies.append(pltpu.async_copy(
            mt_hbm.at[midx_v.at[j]], mrows_v.at[pl.ds(j * CHUNK, CHUNK)], sem))
    for c in copies:
        c.wait()
    pltpu.sync_copy(urows_v, u_out.at[pl.ds(base, BPW)])
    pltpu.sync_copy(mrows_v, m_out.at[pl.ds(base, BPW)])


def _sc_gather(user_table, movie_table, uid2d, mid2d):
    mesh = plsc.VectorSubcoreMesh(
        core_axis_name="c", subcore_axis_name="s",
        num_cores=NC, num_subcores=NS)
    fn = pl.kernel(
        _gather_body,
        mesh=mesh,
        compiler_params=pltpu.CompilerParams(use_tc_tiling_on_sc=False),
        out_type=[
            jax.ShapeDtypeStruct((B, D), jnp.float32),
            jax.ShapeDtypeStruct((B, D), jnp.float32),
        ],
        scratch_types=[
            pltpu.VMEM((NCHUNK, CHUNK), jnp.int32),
            pltpu.VMEM((NCHUNK, CHUNK), jnp.int32),
            pltpu.VMEM((BPW, D), jnp.float32),
            pltpu.VMEM((BPW, D), jnp.float32),
            pltpu.SemaphoreType.DMA,
        ],
    )
    return fn(user_table, movie_table, uid2d, mid2d)


def _mlp_body(u_ref, m_ref, w1t_ref, b1_ref, w2t_ref, b2_ref, w3_ref,
              b3_ref, o_ref):
    x = jnp.concatenate([u_ref[...], m_ref[...]], axis=1)
    h = jnp.dot(x, w1t_ref[...], preferred_element_type=jnp.float32)
    h = jnp.maximum(h + b1_ref[...], 0.0)
    h = jnp.dot(h, w2t_ref[...], preferred_element_type=jnp.float32)
    h = jnp.maximum(h + b2_ref[...], 0.0)
    o_ref[...] = (jnp.sum(h * w3_ref[...], axis=1, keepdims=True)
                  + b3_ref[...])


def _tc_mlp(u, m, w1t, b1r, w2t, b2r, w3, b3r):
    grid = (B // BLK,)
    return pl.pallas_call(
        _mlp_body,
        grid=grid,
        in_specs=[
            pl.BlockSpec((BLK, D), lambda i: (i, 0)),
            pl.BlockSpec((BLK, D), lambda i: (i, 0)),
            pl.BlockSpec((2 * D, HID1), lambda i: (0, 0)),
            pl.BlockSpec((1, HID1), lambda i: (0, 0)),
            pl.BlockSpec((HID1, HID2), lambda i: (0, 0)),
            pl.BlockSpec((1, HID2), lambda i: (0, 0)),
            pl.BlockSpec((1, HID2), lambda i: (0, 0)),
            pl.BlockSpec((1, 1), lambda i: (0, 0)),
        ],
        out_specs=pl.BlockSpec((BLK, 1), lambda i: (i, 0)),
        out_shape=jax.ShapeDtypeStruct((B, 1), jnp.float32),
    )(u, m, w1t, b1r, w2t, b2r, w3, b3r)


@jax.jit
def kernel(user_ids, movie_ids, user_table, movie_table,
           W1, b1, W2, b2, W3, b3):
    uid2d = user_ids.astype(jnp.int32).reshape(B // CHUNK, CHUNK)
    mid2d = movie_ids.astype(jnp.int32).reshape(B // CHUNK, CHUNK)
    u, m = _sc_gather(user_table, movie_table, uid2d, mid2d)
    out = _tc_mlp(u, m, W1.T, b1.reshape(1, HID1), W2.T,
                  b2.reshape(1, HID2), W3, b3.reshape(1, 1))
    return out[:, 0]


# speedup vs baseline: 1.5680x; 1.5680x over previous
import functools

import jax
import jax.numpy as jnp
from jax import lax
from jax.experimental import pallas as pl
from jax.experimental.pallas import tpu as pltpu
from jax.experimental.pallas import tpu_sc as plsc

NC = 2
NS = 16
NW = NC * NS
L = 16

B = 16384
D = 64
CHUNK = 128
BPW = B // NW
NCHUNK = BPW // CHUNK

HID1 = 256
HID2 = 128
BLK = 2048


def _gather_body(ut_hbm, mt_hbm, uid_hbm, mid_hbm, u_out, m_out,
                 uidx_v, midx_v, ub0, ub1, mb0, mb1, sem, osem):
    wid = lax.axis_index("s") * NC + lax.axis_index("c")
    base = wid * BPW
    pltpu.sync_copy(uid_hbm.at[pl.ds(base, BPW)], uidx_v)
    pltpu.sync_copy(mid_hbm.at[pl.ds(base, BPW)], midx_v)

    ubufs = (ub0, ub1)
    mbufs = (mb0, mb1)
    gpc = CHUNK // L

    def issue_chunk(c, ub, mb):
        def issue(g, _):
            uvec = uidx_v[pl.ds((c * gpc + g) * L, L)]
            mvec = midx_v[pl.ds((c * gpc + g) * L, L)]
            for lane in range(L):
                k = g * L + lane
                pltpu.async_copy(ut_hbm.at[pl.ds(uvec[lane], 1)],
                                 ub.at[pl.ds(k, 1)], sem)
                pltpu.async_copy(mt_hbm.at[pl.ds(mvec[lane], 1)],
                                 mb.at[pl.ds(k, 1)], sem)
            return 0
        lax.fori_loop(0, gpc, issue, 0)

    for c in range(NCHUNK):
        ub, mb = ubufs[c % 2], mbufs[c % 2]
        if c >= 2:
            pltpu.make_async_copy(ub, u_out.at[pl.ds(0, CHUNK)], osem).wait()
            pltpu.make_async_copy(mb, m_out.at[pl.ds(0, CHUNK)], osem).wait()
        issue_chunk(c, ub, mb)
        pltpu.make_async_copy(ut_hbm.at[pl.ds(0, CHUNK)], ub, sem).wait()
        pltpu.make_async_copy(mt_hbm.at[pl.ds(0, CHUNK)], mb, sem).wait()
        off = base + c * CHUNK
        pltpu.async_copy(ub, u_out.at[pl.ds(off, CHUNK)], osem)
        pltpu.async_copy(mb, m_out.at[pl.ds(off, CHUNK)], osem)

    for c in (NCHUNK - 2, NCHUNK - 1):
        ub, mb = ubufs[c % 2], mbufs[c % 2]
        pltpu.make_async_copy(ub, u_out.at[pl.ds(0, CHUNK)], osem).wait()
        pltpu.make_async_copy(mb, m_out.at[pl.ds(0, CHUNK)], osem).wait()


def _sc_gather(user_table, movie_table, uid2d, mid2d):
    mesh = plsc.VectorSubcoreMesh(
        core_axis_name="c", subcore_axis_name="s",
        num_cores=NC, num_subcores=NS)
    fn = pl.kernel(
        _gather_body,
        mesh=mesh,
        out_type=[
            jax.ShapeDtypeStruct((B, D), jnp.float32),
            jax.ShapeDtypeStruct((B, D), jnp.float32),
        ],
        scratch_types=[
            pltpu.VMEM((BPW,), jnp.int32),
            pltpu.VMEM((BPW,), jnp.int32),
            pltpu.VMEM((CHUNK, D), jnp.float32),
            pltpu.VMEM((CHUNK, D), jnp.float32),
            pltpu.VMEM((CHUNK, D), jnp.float32),
            pltpu.VMEM((CHUNK, D), jnp.float32),
            pltpu.SemaphoreType.DMA,
            pltpu.SemaphoreType.DMA,
        ],
    )
    return fn(user_table, movie_table, uid2d, mid2d)


def _mlp_body(u_ref, m_ref, w1t_ref, b1_ref, w2t_ref, b2_ref, w3_ref,
              b3_ref, o_ref):
    x = jnp.concatenate([u_ref[...], m_ref[...]], axis=1)
    h = jnp.dot(x, w1t_ref[...], preferred_element_type=jnp.float32)
    h = jnp.maximum(h + b1_ref[...], 0.0)
    h = jnp.dot(h, w2t_ref[...], preferred_element_type=jnp.float32)
    h = jnp.maximum(h + b2_ref[...], 0.0)
    o_ref[...] = (jnp.sum(h * w3_ref[...], axis=1, keepdims=True)
                  + b3_ref[...])


def _tc_mlp(u, m, w1t, b1r, w2t, b2r, w3, b3r):
    grid = (B // BLK,)
    return pl.pallas_call(
        _mlp_body,
        grid=grid,
        in_specs=[
            pl.BlockSpec((BLK, D), lambda i: (i, 0)),
            pl.BlockSpec((BLK, D), lambda i: (i, 0)),
            pl.BlockSpec((2 * D, HID1), lambda i: (0, 0)),
            pl.BlockSpec((1, HID1), lambda i: (0, 0)),
            pl.BlockSpec((HID1, HID2), lambda i: (0, 0)),
            pl.BlockSpec((1, HID2), lambda i: (0, 0)),
            pl.BlockSpec((1, HID2), lambda i: (0, 0)),
            pl.BlockSpec((1, 1), lambda i: (0, 0)),
        ],
        out_specs=pl.BlockSpec((BLK, 1), lambda i: (i, 0)),
        out_shape=jax.ShapeDtypeStruct((B, 1), jnp.float32),
    )(u, m, w1t, b1r, w2t, b2r, w3, b3r)


@jax.jit
def kernel(user_ids, movie_ids, user_table, movie_table,
           W1, b1, W2, b2, W3, b3):
    uid2d = user_ids.astype(jnp.int32)
    mid2d = movie_ids.astype(jnp.int32)
    u, m = _sc_gather(user_table, movie_table, uid2d, mid2d)
    out = _tc_mlp(u, m, W1.T, b1.reshape(1, HID1), W2.T,
                  b2.reshape(1, HID2), W3, b3.reshape(1, 1))
    return out[:, 0]


# speedup vs baseline: 2.6780x; 1.7079x over previous
import jax
import jax.numpy as jnp
from jax import lax
from jax.experimental import pallas as pl
from jax.experimental.pallas import tpu as pltpu
from jax.experimental.pallas import tpu_sc as plsc

NC = 2
NS = 16
NW = NC * NS
L = 16

B = 16384
D = 64
BPW = B // NW
NGRP = BPW // L
NBUF = 8

HID1 = 256
HID2 = 128
BLK = 2048


def _gather_body(ut_hbm, mt_hbm, uid_hbm, mid_hbm, u_out, m_out,
                 idx_v, rows, *blks_and_sems):
    blks = blks_and_sems[:NBUF]
    sems = blks_and_sems[NBUF:]
    wid = lax.axis_index("s") * NC + lax.axis_index("c")
    base = wid * BPW
    iota16 = lax.iota(jnp.int32, L)

    def one_table(tab_hbm, id_hbm, out_hbm):
        pltpu.sync_copy(id_hbm.at[pl.ds(base, BPW)], idx_v)

        def issue(vec, lane, slot):
            c = (vec[lane] // 128) * 128
            pltpu.async_copy(tab_hbm.at[:, pl.ds(c, 128)],
                             blks[slot], sems[slot])

        def extract(vec, g, lane, slot):
            pltpu.make_async_copy(tab_hbm.at[:, pl.ds(0, 128)],
                                  blks[slot], sems[slot]).wait()
            lv = jnp.full((L,), vec[lane] % 128, jnp.int32)
            for j in range(D // L):
                col = plsc.load_gather(blks[slot], [iota16 + j * L, lv])
                rows[pl.ds((g * L + lane) * D + j * L, L)] = col

        vec0 = idx_v[pl.ds(0, L)]
        for lane in range(NBUF):
            issue(vec0, lane, lane)

        def grp(g, vec):
            nxt = idx_v[pl.ds((g + 1) * L, L)]
            for lane in range(NBUF):
                extract(vec, g, lane, lane)
                issue(vec, lane + NBUF, lane)
            for lane in range(NBUF, L):
                extract(vec, g, lane, lane - NBUF)
                issue(nxt, lane - NBUF, lane - NBUF)
            return nxt

        vlast = lax.fori_loop(0, NGRP - 1, grp, vec0)
        g = NGRP - 1
        for lane in range(NBUF):
            extract(vlast, g, lane, lane)
            issue(vlast, lane + NBUF, lane)
        for lane in range(NBUF, L):
            extract(vlast, g, lane, lane - NBUF)
        pltpu.sync_copy(rows, out_hbm.at[pl.ds(base * D, BPW * D)])

    one_table(ut_hbm, uid_hbm, u_out)
    one_table(mt_hbm, mid_hbm, m_out)


def _sc_gather(ut_t, mt_t, uid, mid):
    mesh = plsc.VectorSubcoreMesh(
        core_axis_name="c", subcore_axis_name="s",
        num_cores=NC, num_subcores=NS)
    fn = pl.kernel(
        _gather_body,
        mesh=mesh,
        compiler_params=pltpu.CompilerParams(needs_layout_passes=False),
        out_type=[
            jax.ShapeDtypeStruct((B * D,), jnp.float32),
            jax.ShapeDtypeStruct((B * D,), jnp.float32),
        ],
        scratch_types=(
            [pltpu.VMEM((BPW,), jnp.int32),
             pltpu.VMEM((BPW * D,), jnp.float32)]
            + [pltpu.VMEM((D, 128), jnp.float32)] * NBUF
            + [pltpu.SemaphoreType.DMA] * NBUF
        ),
    )
    return fn(ut_t, mt_t, uid, mid)


def _mlp_body(u_ref, m_ref, w1t_ref, b1_ref, w2t_ref, b2_ref, w3_ref,
              b3_ref, o_ref):
    x = jnp.concatenate([u_ref[...], m_ref[...]], axis=1)
    h = jnp.dot(x, w1t_ref[...], preferred_element_type=jnp.float32)
    h = jnp.maximum(h + b1_ref[...], 0.0)
    h = jnp.dot(h, w2t_ref[...], preferred_element_type=jnp.float32)
    h = jnp.maximum(h + b2_ref[...], 0.0)
    o_ref[...] = (jnp.sum(h * w3_ref[...], axis=1, keepdims=True)
                  + b3_ref[...])


def _tc_mlp(u, m, w1t, b1r, w2t, b2r, w3, b3r):
    grid = (B // BLK,)
    return pl.pallas_call(
        _mlp_body,
        grid=grid,
        in_specs=[
            pl.BlockSpec((BLK, D), lambda i: (i, 0)),
            pl.BlockSpec((BLK, D), lambda i: (i, 0)),
            pl.BlockSpec((2 * D, HID1), lambda i: (0, 0)),
            pl.BlockSpec((1, HID1), lambda i: (0, 0)),
            pl.BlockSpec((HID1, HID2), lambda i: (0, 0)),
            pl.BlockSpec((1, HID2), lambda i: (0, 0)),
            pl.BlockSpec((1, HID2), lambda i: (0, 0)),
            pl.BlockSpec((1, 1), lambda i: (0, 0)),
        ],
        out_specs=pl.BlockSpec((BLK, 1), lambda i: (i, 0)),
        out_shape=jax.ShapeDtypeStruct((B, 1), jnp.float32),
    )(u, m, w1t, b1r, w2t, b2r, w3, b3r)


@jax.jit
def kernel(user_ids, movie_ids, user_table, movie_table,
           W1, b1, W2, b2, W3, b3):
    uid = user_ids.astype(jnp.int32)
    mid = movie_ids.astype(jnp.int32)
    uf, mf = _sc_gather(user_table.T, movie_table.T, uid, mid)
    u = uf.reshape(B, D)
    m = mf.reshape(B, D)
    out = _tc_mlp(u, m, W1.T, b1.reshape(1, HID1), W2.T,
                  b2.reshape(1, HID2), W3, b3.reshape(1, 1))
    return out[:, 0]
